# baseline (speedup 1.0000x reference)
import functools

import jax
import jax.numpy as jnp
from jax import lax
from jax.experimental import pallas as pl
from jax.experimental.pallas import tpu as pltpu
from jax.experimental.pallas import tpu_sc as plsc

D_MODEL = 64
SCALE = 8.0

NC, NS = 2, 16
NW = NC * NS
ROWS, COLS = 4096, 200
B = ROWS * COLS
B_PER_W = B // NW
CHUNK = 128
NCHUNK = B_PER_W // CHUNK

_mesh = plsc.VectorSubcoreMesh(core_axis_name="c", subcore_axis_name="s")


@functools.partial(
    pl.kernel,
    out_type=jax.ShapeDtypeStruct((NW, NCHUNK, CHUNK, D_MODEL), jnp.float32),
    mesh=_mesh,
    scratch_types=[
        pltpu.VMEM((NCHUNK, CHUNK), jnp.int32),
        pltpu.VMEM((CHUNK, D_MODEL), jnp.float32),
        pltpu.SemaphoreType.DMA,
    ],
    compiler_params=pltpu.CompilerParams(use_tc_tiling_on_sc=False),
)
def _emb_lookup(x_hbm, lut_hbm, out_hbm, idx_v, rows_v, sem):
    wid = lax.axis_index("s") * NC + lax.axis_index("c")
    pltpu.sync_copy(x_hbm.at[wid], idx_v)

    @pl.loop(0, NCHUNK)
    def _chunk(j):
        pltpu.async_copy(lut_hbm.at[idx_v.at[j]], rows_v, sem).wait()

        @pl.loop(0, CHUNK)
        def _row(r):
            for c in range(D_MODEL // 16):
                s = pl.ds(c * 16, 16)
                rows_v[r, s] = rows_v[r, s] * SCALE

        pltpu.sync_copy(rows_v, out_hbm.at[wid, j])


def kernel(x, lut):
    xf = x.reshape(NW, NCHUNK, CHUNK)
    out = _emb_lookup(xf, lut)
    return out.reshape(ROWS, COLS, D_MODEL)


# speedup vs baseline: 1.1001x; 1.1001x over previous
import functools

import jax
import jax.numpy as jnp
from jax import lax
from jax.experimental import pallas as pl
from jax.experimental.pallas import tpu as pltpu
from jax.experimental.pallas import tpu_sc as plsc

D_MODEL = 64
SCALE = 8.0

NC, NS = 2, 16
NW = NC * NS
ROWS, COLS = 4096, 200
B = ROWS * COLS
B_PER_W = B // NW
CHUNK = 128
NCHUNK = B_PER_W // CHUNK
NB = 4

_mesh = plsc.VectorSubcoreMesh(core_axis_name="c", subcore_axis_name="s")


@functools.partial(
    pl.kernel,
    out_type=jax.ShapeDtypeStruct((NW, NCHUNK, CHUNK, D_MODEL), jnp.float32),
    mesh=_mesh,
    scratch_types=[
        pltpu.VMEM((NCHUNK, CHUNK), jnp.int32),
        pltpu.VMEM((NB, CHUNK, D_MODEL), jnp.float32),
        pltpu.VMEM((NB, CHUNK, D_MODEL), jnp.float32),
    ] + [pltpu.SemaphoreType.DMA] * (2 * NB),
    compiler_params=pltpu.CompilerParams(use_tc_tiling_on_sc=False),
)
def _emb_lookup(x_hbm, lut_hbm, out_hbm, idx_v, gbuf, obuf, *sems):
    gsem, wsem = sems[:NB], sems[NB:]
    wid = lax.axis_index("s") * NC + lax.axis_index("c")
    pltpu.sync_copy(x_hbm.at[wid], idx_v)

    for b in range(NB):
        pltpu.async_copy(lut_hbm.at[idx_v.at[b]], gbuf.at[b], gsem[b])

    @pl.loop(0, NCHUNK, step=NB)
    def _grp(j0):
        for b in range(NB):
            j = j0 + b

            pltpu.make_async_copy(
                lut_hbm.at[idx_v.at[j]], gbuf.at[b], gsem[b]).wait()

            @pl.when(j >= NB)
            def _wait_write():
                pltpu.make_async_copy(
                    obuf.at[b], out_hbm.at[wid, j - NB], wsem[b]).wait()

            @pl.loop(0, CHUNK, unroll=8)
            def _row(r):
                for c in range(D_MODEL // 16):
                    s = pl.ds(c * 16, 16)
                    obuf[b, r, s] = gbuf[b, r, s] * SCALE

            pltpu.async_copy(obuf.at[b], out_hbm.at[wid, j], wsem[b])

            @pl.when(j + NB < NCHUNK)
            def _next_gather():
                pltpu.async_copy(
                    lut_hbm.at[idx_v.at[j + NB]], gbuf.at[b], gsem[b])

    for b in range(NB):
        pltpu.make_async_copy(
            obuf.at[b], out_hbm.at[wid, NCHUNK - NB + b], wsem[b]).wait()


def kernel(x, lut):
    xf = x.reshape(NW, NCHUNK, CHUNK)
    out = _emb_lookup(xf, lut)
    return out.reshape(ROWS, COLS, D_MODEL)
